# baseline (speedup 1.0000x reference)
import functools

import jax
import jax.numpy as jnp
from jax import lax
from jax.experimental import pallas as pl
from jax.experimental.pallas import tpu as pltpu
from jax.experimental.pallas import tpu_sc as plsc

N, D, E = 10000, 128, 320000
NC, NS = 2, 16
NW = NC * NS
EPT = E // NW
CH = 80
NJ = EPT // CH
NB = 4
NTRIP = NJ // NB
NPAD = 10240
RPT = NPAD // NS


def _sc_agg_body(x_hbm, src_hbm, dst_hbm, zero_hbm, out_hbm,
                 agg_sh, srcs, dsts, rows, gsems, ssems, isems, dsems):
    c = lax.axis_index("c")
    s = lax.axis_index("s")
    wid = s * NC + c

    pltpu.sync_copy(zero_hbm, agg_sh.at[pl.ds(s * RPT, RPT)])
    plsc.subcore_barrier()

    ebase = wid * EPT

    def load_src(j, k):
        pltpu.async_copy(src_hbm.at[pl.ds(ebase + j * CH, CH)],
                         srcs.at[k], isems.at[k])

    def load_dst(j, k):
        pltpu.async_copy(dst_hbm.at[pl.ds(ebase + j * CH, CH)],
                         dsts.at[k], dsems.at[k])

    def gather(k):
        return pltpu.async_copy(x_hbm.at[srcs.at[k]], rows.at[k],
                                gsems.at[k])

    def scatter(k):
        return pltpu.async_copy(rows.at[k], agg_sh.at[dsts.at[k]],
                                ssems.at[k], add=True)

    def wait_sem(ref, sem):
        pltpu.make_async_copy(dst_hbm.at[pl.ds(0, CH)], ref, sem).wait()

    def wait_rows(k):
        pltpu.make_async_copy(x_hbm.at[srcs.at[k]], rows.at[k],
                              gsems.at[k]).wait()

    for k in range(NB):
        load_src(k, k)
        load_dst(k, k)
    for k in range(NB):
        wait_sem(srcs.at[k], isems.at[k])
        gather(k)

    def trip(t, carry):
        j = t * NB
        scs = []
        for k in range(NB):
            wait_rows(k)
            wait_sem(dsts.at[k], dsems.at[k])
            scs.append(scatter(k))
            if k < NJ % NB:
                load_src(j + k + NB, k)
            else:
                @pl.when(t < NTRIP - 1)
                def _presrc(k=k, nxt=j + k + NB):
                    load_src(nxt, k)
        for k in range(NB):
            scs[k].wait()
            if k < NJ % NB:
                load_dst(j + k + NB, k)
                wait_sem(srcs.at[k], isems.at[k])
                gather(k)
            else:
                @pl.when(t < NTRIP - 1)
                def _prefetch(k=k, nxt=j + k + NB):
                    load_dst(nxt, k)
                    wait_sem(srcs.at[k], isems.at[k])
                    gather(k)
        return carry

    lax.fori_loop(0, NTRIP, trip, 0)

    for k in range(NJ % NB):
        wait_rows(k)
        wait_sem(dsts.at[k], dsems.at[k])
        scatter(k).wait()

    plsc.subcore_barrier()
    pltpu.sync_copy(agg_sh.at[pl.ds(s * RPT, RPT)],
                    out_hbm.at[c, pl.ds(s * RPT, RPT)])


@functools.cache
def _sc_agg():
    return pl.kernel(
        _sc_agg_body,
        mesh=plsc.VectorSubcoreMesh(core_axis_name="c", subcore_axis_name="s"),
        out_type=jax.ShapeDtypeStruct((NC, NPAD, D), jnp.float32),
        scratch_types=[
            pltpu.VMEM_SHARED((NPAD, D), jnp.float32),
            pltpu.VMEM((NB, CH), jnp.int32),
            pltpu.VMEM((NB, CH), jnp.int32),
            pltpu.VMEM((NB, CH, D), jnp.float32),
            pltpu.SemaphoreType.DMA((NB,)),
            pltpu.SemaphoreType.DMA((NB,)),
            pltpu.SemaphoreType.DMA((NB,)),
            pltpu.SemaphoreType.DMA((NB,)),
        ],
    )


BLK = 1000
NBLK = N // BLK


def _tc_body(x_ref, p_ref, wt_ref, b_ref, g_ref, bt_ref, eps_ref, o_ref,
             h_scr, sum_scr, sq_scr):
    phase = pl.program_id(0)
    blk = pl.program_id(1)
    x = x_ref[...]

    @pl.when(phase == 0)
    def _compute():
        u = (1.0 + eps_ref[0, 0]) * x + p_ref[0] + p_ref[1]
        h = jnp.dot(u, wt_ref[...],
                    preferred_element_type=jnp.float32) + b_ref[...]
        h_scr[pl.ds(blk * BLK, BLK), :] = h

        @pl.when(blk == 0)
        def _init():
            sum_scr[...] = jnp.zeros_like(sum_scr)
            sq_scr[...] = jnp.zeros_like(sq_scr)

        sum_scr[...] += jnp.sum(h, axis=0, keepdims=True)
        sq_scr[...] += jnp.sum(h * h, axis=0, keepdims=True)

    @pl.when(phase == 1)
    def _normalize():
        mean = sum_scr[...] * (1.0 / N)
        var = sq_scr[...] * (1.0 / N) - mean * mean
        h = h_scr[pl.ds(blk * BLK, BLK), :]
        hn = (h - mean) * lax.rsqrt(var + 1e-5) * g_ref[...] + bt_ref[...]
        o_ref[...] = jnp.maximum(hn, 0.0) + x


def kernel(x, edge_index, W, b, eps, gamma, beta):
    partials = _sc_agg()(x, edge_index[0], edge_index[1],
                         jnp.zeros((RPT, D), jnp.float32))
    return pl.pallas_call(
        _tc_body,
        grid=(2, NBLK),
        in_specs=[
            pl.BlockSpec((BLK, D), lambda p, i: (i, 0)),
            pl.BlockSpec((NC, BLK, D), lambda p, i: (0, i, 0)),
            pl.BlockSpec((D, D), lambda p, i: (0, 0)),
            pl.BlockSpec((1, D), lambda p, i: (0, 0)),
            pl.BlockSpec((1, D), lambda p, i: (0, 0)),
            pl.BlockSpec((1, D), lambda p, i: (0, 0)),
            pl.BlockSpec((1, 1), lambda p, i: (0, 0)),
        ],
        out_specs=pl.BlockSpec((BLK, D), lambda p, i: (i, 0)),
        out_shape=jax.ShapeDtypeStruct((N, D), jnp.float32),
        scratch_shapes=[
            pltpu.VMEM((N, D), jnp.float32),
            pltpu.VMEM((1, D), jnp.float32),
            pltpu.VMEM((1, D), jnp.float32),
        ],
    )(x, partials, W.T,
      b.reshape(1, D), gamma.reshape(1, D), beta.reshape(1, D),
      eps.reshape(1, 1))


# speedup vs baseline: 1.0687x; 1.0687x over previous
import functools

import jax
import jax.numpy as jnp
from jax import lax
from jax.experimental import pallas as pl
from jax.experimental.pallas import tpu as pltpu
from jax.experimental.pallas import tpu_sc as plsc

N, D, E = 10000, 128, 320000
NC, NS = 2, 16
NW = NC * NS
EPT = E // NW
CH = 80
NJ = EPT // CH
NB = 4
NTRIP = NJ // NB
NPAD = 10240
RPT = NPAD // NS


def _sc_agg_body(x_hbm, src_hbm, dst_hbm, zero_hbm, out_hbm,
                 agg_sh, srcs, dsts, rows, gsems, ssems, isems, dsems):
    c = lax.axis_index("c")
    s = lax.axis_index("s")
    wid = s * NC + c

    pltpu.sync_copy(zero_hbm, agg_sh.at[pl.ds(s * RPT, RPT)])
    plsc.subcore_barrier()

    ebase = wid * EPT

    def load_src(j, k):
        pltpu.async_copy(src_hbm.at[pl.ds(ebase + j * CH, CH)],
                         srcs.at[k], isems.at[k])

    def load_dst(j, k):
        pltpu.async_copy(dst_hbm.at[pl.ds(ebase + j * CH, CH)],
                         dsts.at[k], dsems.at[k])

    def gather(k):
        return pltpu.async_copy(x_hbm.at[srcs.at[k]], rows.at[k],
                                gsems.at[k])

    def scatter(k):
        return pltpu.async_copy(rows.at[k], agg_sh.at[dsts.at[k]],
                                ssems.at[k], add=True)

    def wait_sem(ref, sem):
        pltpu.make_async_copy(dst_hbm.at[pl.ds(0, CH)], ref, sem).wait()

    def wait_rows(k):
        pltpu.make_async_copy(x_hbm.at[srcs.at[k]], rows.at[k],
                              gsems.at[k]).wait()

    for k in range(NB):
        load_src(k, k)
        load_dst(k, k)
    for k in range(NB):
        wait_sem(srcs.at[k], isems.at[k])
        gather(k)

    def trip(t, carry):
        j = t * NB
        scs = []
        for k in range(NB):
            wait_rows(k)
            wait_sem(dsts.at[k], dsems.at[k])
            scs.append(scatter(k))
            if k < NJ % NB:
                load_src(j + k + NB, k)
            else:
                @pl.when(t < NTRIP - 1)
                def _presrc(k=k, nxt=j + k + NB):
                    load_src(nxt, k)
        for k in range(NB):
            scs[k].wait()
            if k < NJ % NB:
                load_dst(j + k + NB, k)
                wait_sem(srcs.at[k], isems.at[k])
                gather(k)
            else:
                @pl.when(t < NTRIP - 1)
                def _prefetch(k=k, nxt=j + k + NB):
                    load_dst(nxt, k)
                    wait_sem(srcs.at[k], isems.at[k])
                    gather(k)
        return carry

    lax.fori_loop(0, NTRIP, trip, 0)

    for k in range(NJ % NB):
        wait_rows(k)
        wait_sem(dsts.at[k], dsems.at[k])
        scatter(k).wait()

    plsc.subcore_barrier()
    pltpu.sync_copy(agg_sh.at[pl.ds(s * RPT, RPT)],
                    out_hbm.at[c, pl.ds(s * RPT, RPT)])


@functools.cache
def _sc_agg():
    return pl.kernel(
        _sc_agg_body,
        mesh=plsc.VectorSubcoreMesh(core_axis_name="c", subcore_axis_name="s"),
        out_type=jax.ShapeDtypeStruct((NC, NPAD, D), jnp.float32),
        scratch_types=[
            pltpu.VMEM_SHARED((NPAD, D), jnp.float32),
            pltpu.VMEM((NB, CH), jnp.int32),
            pltpu.VMEM((NB, CH), jnp.int32),
            pltpu.VMEM((NB, CH, D), jnp.float32),
            pltpu.SemaphoreType.DMA((NB,)),
            pltpu.SemaphoreType.DMA((NB,)),
            pltpu.SemaphoreType.DMA((NB,)),
            pltpu.SemaphoreType.DMA((NB,)),
        ],
    )


def _tc_body(x_ref, p_ref, wt_ref, b_ref, g_ref, bt_ref, eps_ref, o_ref):
    x = x_ref[...]
    agg = p_ref[0, :N] + p_ref[1, :N]
    u = (1.0 + eps_ref[0, 0]) * x + agg
    h = jnp.dot(u, wt_ref[...], preferred_element_type=jnp.float32) + b_ref[...]
    mean = jnp.mean(h, axis=0, keepdims=True)
    var = jnp.mean(h * h, axis=0, keepdims=True) - mean * mean
    hn = (h - mean) * lax.rsqrt(var + 1e-5) * g_ref[...] + bt_ref[...]
    o_ref[...] = jnp.maximum(hn, 0.0) + x


def kernel(x, edge_index, W, b, eps, gamma, beta):
    partials = _sc_agg()(x, edge_index[0], edge_index[1],
                         jnp.zeros((RPT, D), jnp.float32))
    return pl.pallas_call(
        _tc_body,
        out_shape=jax.ShapeDtypeStruct((N, D), jnp.float32),
    )(x, partials, W.T,
      b.reshape(1, D), gamma.reshape(1, D), beta.reshape(1, D),
      eps.reshape(1, 1))
